# Initial kernel scaffold; baseline (speedup 1.0000x reference)
import jax
import jax.numpy as jnp
from jax.experimental import pallas as pl


def kernel(vertices, rows, cols, vals):
    raise NotImplementedError("write your pallas kernel here")


# speedup vs baseline: 3.3355x; 3.3355x over previous
import functools

import jax
import jax.numpy as jnp
from jax import lax
from jax.experimental import pallas as pl
from jax.experimental.pallas import tpu as pltpu
from jax.experimental.pallas import tpu_sc as plsc

_NC = 2
_NS = 16
_L = 16
_CHUNK = 128


@functools.lru_cache(maxsize=None)
def _build(nnz_pad, n_hd, d, chunks_per_tile):
    half = n_hd // _NC
    rows_per_tile = half // _NS
    d_vregs = d // _L

    def body(cols_hbm, rows_hbm, vals_hbm, m2_hbm, out_hbm,
             colbuf, rowbuf, valbuf, gbuf, acc, sem):
        sc = lax.axis_index("c")
        tid = lax.axis_index("s")

        def zero_row(i, _):
            for j in range(d_vregs):
                gbuf[i, pl.ds(j * _L, _L)] = jnp.zeros((_L,), jnp.float32)
            return 0
        lax.fori_loop(0, _CHUNK, zero_row, 0)
        for k in range(rows_per_tile // _CHUNK):
            pltpu.sync_copy(
                gbuf, acc.at[pl.ds(tid * rows_per_tile + k * _CHUNK, _CHUNK)])
        plsc.subcore_barrier()

        def chunk_body(cidx, _):
            base = (tid * chunks_per_tile + cidx) * _CHUNK
            pltpu.sync_copy(cols_hbm.at[pl.ds(base, _CHUNK)], colbuf)
            pltpu.sync_copy(rows_hbm.at[pl.ds(base, _CHUNK)], rowbuf)
            pltpu.sync_copy(vals_hbm.at[pl.ds(base, _CHUNK)], valbuf)
            pltpu.async_copy(m2_hbm.at[colbuf], gbuf, sem).wait()
            sc_vec = jnp.full((_L,), sc, jnp.int32)
            for i in range(_CHUNK // _L):
                sl = pl.ds(i * _L, _L)
                r = rowbuf[sl]
                v = valbuf[sl]
                mine = lax.shift_right_logical(r, 13) == sc_vec
                valbuf[sl] = jnp.where(mine, v, jnp.zeros((_L,), jnp.float32))
                rowbuf[sl] = lax.bitwise_and(r, half - 1)
            def scale(g, _):
                v16 = valbuf[pl.ds(g * _L, _L)]
                for i in range(_L):
                    v = v16[i]
                    row = g * _L + i
                    for j in range(d_vregs):
                        s = pl.ds(j * _L, _L)
                        gbuf[row, s] = gbuf[row, s] * v
                return 0
            lax.fori_loop(0, _CHUNK // _L, scale, 0)
            pltpu.sync_copy(gbuf, acc.at[rowbuf], add=True)
            return 0

        lax.fori_loop(0, chunks_per_tile, chunk_body, 0)
        plsc.subcore_barrier()
        for k in range(rows_per_tile // _CHUNK):
            off = tid * rows_per_tile + k * _CHUNK
            pltpu.sync_copy(acc.at[pl.ds(off, _CHUNK)],
                            out_hbm.at[pl.ds(sc * half + off, _CHUNK)])

    return pl.kernel(
        body,
        out_type=jax.ShapeDtypeStruct((n_hd, d), jnp.float32),
        mesh=plsc.VectorSubcoreMesh(core_axis_name="c", subcore_axis_name="s"),
        scratch_types=[
            pltpu.VMEM((_CHUNK,), jnp.int32),
            pltpu.VMEM((_CHUNK,), jnp.int32),
            pltpu.VMEM((_CHUNK,), jnp.float32),
            pltpu.VMEM((_CHUNK, d), jnp.float32),
            pltpu.VMEM_SHARED((n_hd // _NC, d), jnp.float32),
            pltpu.SemaphoreType.DMA,
        ],
        compiler_params=pltpu.CompilerParams(use_tc_tiling_on_sc=False),
    )


def kernel(vertices, rows, cols, vals):
    if vertices.ndim != 3:
        vertices = vertices[None, :, :]
    b, m, k = vertices.shape
    d = b * k
    n_hd = m
    m2 = jnp.transpose(vertices, (1, 0, 2)).reshape(m, d)

    nnz = rows.shape[0]
    per_tile = _NS * _CHUNK
    chunks_per_tile = -(-nnz // per_tile)
    nnz_pad = chunks_per_tile * per_tile
    pad = nnz_pad - nnz
    rows_p = jnp.concatenate([rows.astype(jnp.int32),
                              jnp.zeros((pad,), jnp.int32)])
    cols_p = jnp.concatenate([cols.astype(jnp.int32),
                              jnp.zeros((pad,), jnp.int32)])
    vals_p = jnp.concatenate([vals, jnp.zeros((pad,), jnp.float32)])

    out = _build(nnz_pad, n_hd, d, chunks_per_tile)(cols_p, rows_p, vals_p, m2)
    return jnp.transpose(out.reshape(n_hd, b, k), (1, 0, 2)).astype(jnp.float32)


# speedup vs baseline: 4.7573x; 1.4263x over previous
import functools

import jax
import jax.numpy as jnp
from jax import lax
from jax.experimental import pallas as pl
from jax.experimental.pallas import tpu as pltpu
from jax.experimental.pallas import tpu_sc as plsc

_NC = 2
_NS = 16
_L = 16
_CHUNK = 128
_SBLEN = 1536
_RS = 512


@functools.lru_cache(maxsize=None)
def _build(nnz_pad, n_hd, d, per_tile):
    half = n_hd // _NC
    half_shift = half.bit_length() - 1
    rows_per_tile = half // _NS
    d_vregs = d // _L
    nsb = per_tile // _SBLEN

    def body(cols_hbm, rows_hbm, vals_hbm, m2_hbm, out_hbm,
             cstage, rstage, vstage, colr, rowr, valr,
             colloc, rowloc, valloc, gbuf, acc, sem):
        sc = lax.axis_index("c")
        tid = lax.axis_index("s")
        sc_vec = jnp.full((_L,), sc, jnp.int32)
        zero_f = jnp.zeros((_L,), jnp.float32)
        zero_i = jnp.zeros((_L,), jnp.int32)

        def zero_row(i, _):
            for j in range(d_vregs):
                gbuf[i, pl.ds(j * _L, _L)] = zero_f
            return 0
        lax.fori_loop(0, _CHUNK, zero_row, 0)
        for k in range(rows_per_tile // _CHUNK):
            pltpu.sync_copy(
                gbuf, acc.at[pl.ds(tid * rows_per_tile + k * _CHUNK, _CHUNK)])

        def process_chunk(done):
            base = lax.bitwise_and(done, _RS - 1)
            for g in range(_CHUNK // _L):
                sl = pl.ds(g * _L, _L)
                src = pl.ds(base + g * _L, _L)
                colloc[sl] = colr[src]
                rowloc[sl] = rowr[src]
                valloc[sl] = valr[src]
            pltpu.async_copy(m2_hbm.at[colloc], gbuf, sem).wait()

            def scale(g, _):
                v16 = valloc[pl.ds(g * _L, _L)]
                for i in range(_L):
                    v = v16[i]
                    row = g * _L + i
                    for j in range(d_vregs):
                        s = pl.ds(j * _L, _L)
                        gbuf[row, s] = gbuf[row, s] * v
                return 0
            lax.fori_loop(0, _CHUNK // _L, scale, 0)
            pltpu.sync_copy(gbuf, acc.at[rowloc], add=True)

        def superblock(sb, carry):
            base = tid * per_tile + sb * _SBLEN
            pltpu.sync_copy(cols_hbm.at[pl.ds(base, _SBLEN)], cstage)
            pltpu.sync_copy(rows_hbm.at[pl.ds(base, _SBLEN)], rstage)
            pltpu.sync_copy(vals_hbm.at[pl.ds(base, _SBLEN)], vstage)

            def grp(g, carry):
                cnt, done = carry
                sl = pl.ds(g * _L, _L)
                r = rstage[sl]
                mine = lax.shift_right_logical(r, half_shift) == sc_vec
                incl = plsc.cumsum(mine.astype(jnp.int32))
                pos = lax.bitwise_and(
                    incl + jnp.full((_L,), cnt - 1, jnp.int32), _RS - 1)
                plsc.store_scatter(colr, [pos], cstage[sl], mask=mine)
                plsc.store_scatter(rowr, [pos],
                                   lax.bitwise_and(r, half - 1), mask=mine)
                plsc.store_scatter(valr, [pos], vstage[sl], mask=mine)
                cnt = cnt + incl[_L - 1]
                full = (cnt - done) >= _CHUNK

                @pl.when(full)
                def _():
                    process_chunk(done)

                return cnt, done + jnp.where(full, _CHUNK, 0).astype(jnp.int32)

            return lax.fori_loop(0, _SBLEN // _L, grp, carry)

        cnt, done = lax.fori_loop(0, nsb, superblock,
                                  (jnp.int32(0), jnp.int32(0)))

        for k in range(_CHUNK // _L):
            tail = pl.ds(lax.bitwise_and(cnt + k * _L, _RS - 1), _L)
            colr[tail] = zero_i
            rowr[tail] = zero_i
            valr[tail] = zero_f

        @pl.when(cnt > done)
        def _():
            process_chunk(done)

        plsc.subcore_barrier()
        for k in range(rows_per_tile // _CHUNK):
            off = tid * rows_per_tile + k * _CHUNK
            pltpu.sync_copy(acc.at[pl.ds(off, _CHUNK)],
                            out_hbm.at[pl.ds(sc * half + off, _CHUNK)])

    return pl.kernel(
        body,
        out_type=jax.ShapeDtypeStruct((n_hd, d), jnp.float32),
        mesh=plsc.VectorSubcoreMesh(core_axis_name="c", subcore_axis_name="s"),
        scratch_types=[
            pltpu.VMEM((_SBLEN,), jnp.int32),
            pltpu.VMEM((_SBLEN,), jnp.int32),
            pltpu.VMEM((_SBLEN,), jnp.float32),
            pltpu.VMEM((_RS,), jnp.int32),
            pltpu.VMEM((_RS,), jnp.int32),
            pltpu.VMEM((_RS,), jnp.float32),
            pltpu.VMEM((_CHUNK,), jnp.int32),
            pltpu.VMEM((_CHUNK,), jnp.int32),
            pltpu.VMEM((_CHUNK,), jnp.float32),
            pltpu.VMEM((_CHUNK, d), jnp.float32),
            pltpu.VMEM_SHARED((n_hd // _NC, d), jnp.float32),
            pltpu.SemaphoreType.DMA,
        ],
        compiler_params=pltpu.CompilerParams(use_tc_tiling_on_sc=False,
                                             needs_layout_passes=False),
    )


def kernel(vertices, rows, cols, vals):
    if vertices.ndim != 3:
        vertices = vertices[None, :, :]
    b, m, k = vertices.shape
    d = b * k
    n_hd = m
    m2 = jnp.transpose(vertices, (1, 0, 2)).reshape(m, d)

    nnz = rows.shape[0]
    per_tile = -(-nnz // (_NS * _SBLEN)) * _SBLEN
    nnz_pad = per_tile * _NS
    pad = nnz_pad - nnz
    rows_p = jnp.concatenate([rows.astype(jnp.int32),
                              jnp.zeros((pad,), jnp.int32)])
    cols_p = jnp.concatenate([cols.astype(jnp.int32),
                              jnp.zeros((pad,), jnp.int32)])
    vals_p = jnp.concatenate([vals, jnp.zeros((pad,), jnp.float32)])

    out = _build(nnz_pad, n_hd, d, per_tile)(cols_p, rows_p, vals_p, m2)
    return jnp.transpose(out.reshape(n_hd, b, k), (1, 0, 2)).astype(jnp.float32)


# speedup vs baseline: 5.8519x; 1.2301x over previous
import functools

import jax
import jax.numpy as jnp
from jax import lax
from jax.experimental import pallas as pl
from jax.experimental.pallas import tpu as pltpu
from jax.experimental.pallas import tpu_sc as plsc

_NC = 2
_NS = 16
_L = 16
_H = 64
_CHUNK = 2 * _H
_SBLEN = 1536
_RS = 512


@functools.lru_cache(maxsize=None)
def _build(nnz_pad, n_hd, d, per_tile):
    half = n_hd // _NC
    half_shift = half.bit_length() - 1
    rows_per_tile = half // _NS
    d_vregs = d // _L
    nsb = per_tile // _SBLEN

    def body(cols_hbm, rows_hbm, vals_hbm, m2_hbm, out_hbm,
             cstage, rstage, vstage, colr, rowr, valr,
             colloc, rowloc, valloc, gbuf, acc,
             sem_g, sem_s):
        sc = lax.axis_index("c")
        tid = lax.axis_index("s")
        sc_vec = jnp.full((_L,), sc, jnp.int32)
        zero_f = jnp.zeros((_L,), jnp.float32)
        zero_i = jnp.zeros((_L,), jnp.int32)

        def zero_row(i, _):
            for j in range(d_vregs):
                gbuf[i, pl.ds(j * _L, _L)] = zero_f
            return 0
        lax.fori_loop(0, _CHUNK, zero_row, 0)
        for k in range(rows_per_tile // _CHUNK):
            pltpu.sync_copy(
                gbuf, acc.at[pl.ds(tid * rows_per_tile + k * _CHUNK, _CHUNK)])
        plsc.subcore_barrier()

        for h in range(2):
            for g in range(_H // _L):
                sl = pl.ds(h * _H + g * _L, _L)
                colloc[sl] = zero_i
                rowloc[h][pl.ds(g * _L, _L)] = zero_i
                valloc[sl] = zero_f
            pltpu.async_copy(gbuf.at[pl.ds(h * _H, _H)],
                             acc.at[rowloc[h]], add=True, sem=sem_s[h])

        def drain_scatter(h):
            pltpu.make_async_copy(gbuf.at[pl.ds(h * _H, _H)],
                                  acc.at[rowloc[h]], sem_s[h]).wait()

        def fire_gather(h, gath):
            drain_scatter(h)
            base = lax.bitwise_and(gath, _RS - 1)
            for g in range(_H // _L):
                sl = pl.ds(h * _H + g * _L, _L)
                src = pl.ds(base + g * _L, _L)
                colloc[sl] = colr[src]
                rowloc[h][pl.ds(g * _L, _L)] = rowr[src]
                valloc[sl] = valr[src]
            pltpu.async_copy(m2_hbm.at[colloc.at[pl.ds(h * _H, _H)]],
                             gbuf.at[pl.ds(h * _H, _H)], sem_g[h])

        def process_half(h):
            pltpu.make_async_copy(m2_hbm.at[colloc.at[pl.ds(h * _H, _H)]],
                                  gbuf.at[pl.ds(h * _H, _H)], sem_g[h]).wait()

            def scale(g, _):
                v16 = valloc[pl.ds(h * _H + g * _L, _L)]
                for i in range(_L):
                    v = v16[i]
                    row = h * _H + g * _L + i
                    for j in range(d_vregs):
                        s = pl.ds(j * _L, _L)
                        gbuf[row, s] = gbuf[row, s] * v
                return 0
            lax.fori_loop(0, _H // _L, scale, 0)
            pltpu.async_copy(gbuf.at[pl.ds(h * _H, _H)],
                             acc.at[rowloc[h]], add=True, sem=sem_s[h])

        def superblock(sb, carry):
            base = tid * per_tile + sb * _SBLEN
            pltpu.sync_copy(cols_hbm.at[pl.ds(base, _SBLEN)], cstage)
            pltpu.sync_copy(rows_hbm.at[pl.ds(base, _SBLEN)], rstage)
            pltpu.sync_copy(vals_hbm.at[pl.ds(base, _SBLEN)], vstage)

            def grp(g, carry):
                cnt, gath, done = carry
                sl = pl.ds(g * _L, _L)
                r = rstage[sl]
                mine = lax.shift_right_logical(r, half_shift) == sc_vec
                incl = plsc.cumsum(mine.astype(jnp.int32))
                pos = lax.bitwise_and(
                    incl + jnp.full((_L,), cnt - 1, jnp.int32), _RS - 1)
                plsc.store_scatter(colr, [pos], cstage[sl], mask=mine)
                plsc.store_scatter(rowr, [pos],
                                   lax.bitwise_and(r, half - 1), mask=mine)
                plsc.store_scatter(valr, [pos], vstage[sl], mask=mine)
                cnt = cnt + incl[_L - 1]

                fire = (cnt - gath) >= _H
                even = lax.bitwise_and(gath, _H) == 0

                @pl.when(jnp.logical_and(fire, even))
                def _():
                    fire_gather(0, gath)

                @pl.when(jnp.logical_and(fire, jnp.logical_not(even)))
                def _():
                    fire_gather(1, gath)

                gath = gath + jnp.where(fire, _H, 0).astype(jnp.int32)
                proc = (gath - done) >= _CHUNK

                @pl.when(proc)
                def _():
                    process_half(0)
                    process_half(1)

                done = done + jnp.where(proc, _CHUNK, 0).astype(jnp.int32)
                return cnt, gath, done

            return lax.fori_loop(0, _SBLEN // _L, grp, carry)

        cnt, gath, done = lax.fori_loop(
            0, nsb, superblock,
            (jnp.int32(0), jnp.int32(0), jnp.int32(0)))

        for k in range(_CHUNK // _L):
            tail = pl.ds(lax.bitwise_and(cnt + k * _L, _RS - 1), _L)
            colr[tail] = zero_i
            rowr[tail] = zero_i
            valr[tail] = zero_f

        owe = cnt > gath
        even = lax.bitwise_and(gath, _H) == 0

        @pl.when(jnp.logical_and(owe, even))
        def _():
            fire_gather(0, gath)

        @pl.when(jnp.logical_and(owe, jnp.logical_not(even)))
        def _():
            fire_gather(1, gath)

        gath = gath + jnp.where(owe, _H, 0).astype(jnp.int32)

        @pl.when(gath - done >= _CHUNK)
        def _():
            process_half(0)
            process_half(1)

        @pl.when(gath - done == _H)
        def _():
            process_half(0)

        drain_scatter(0)
        drain_scatter(1)

        plsc.subcore_barrier()
        for k in range(rows_per_tile // _CHUNK):
            off = tid * rows_per_tile + k * _CHUNK
            pltpu.sync_copy(acc.at[pl.ds(off, _CHUNK)],
                            out_hbm.at[pl.ds(sc * half + off, _CHUNK)])

    return pl.kernel(
        body,
        out_type=jax.ShapeDtypeStruct((n_hd, d), jnp.float32),
        mesh=plsc.VectorSubcoreMesh(core_axis_name="c", subcore_axis_name="s"),
        scratch_types=[
            pltpu.VMEM((_SBLEN,), jnp.int32),
            pltpu.VMEM((_SBLEN,), jnp.int32),
            pltpu.VMEM((_SBLEN,), jnp.float32),
            pltpu.VMEM((_RS,), jnp.int32),
            pltpu.VMEM((_RS,), jnp.int32),
            pltpu.VMEM((_RS,), jnp.float32),
            pltpu.VMEM((_CHUNK,), jnp.int32),
            [pltpu.VMEM((_H,), jnp.int32)] * 2,
            pltpu.VMEM((_CHUNK,), jnp.float32),
            pltpu.VMEM((_CHUNK, d), jnp.float32),
            pltpu.VMEM_SHARED((n_hd // _NC, d), jnp.float32),
            [pltpu.SemaphoreType.DMA] * 2,
            [pltpu.SemaphoreType.DMA] * 2,
        ],
        compiler_params=pltpu.CompilerParams(use_tc_tiling_on_sc=False,
                                             needs_layout_passes=False),
    )


def kernel(vertices, rows, cols, vals):
    if vertices.ndim != 3:
        vertices = vertices[None, :, :]
    b, m, k = vertices.shape
    d = b * k
    n_hd = m
    m2 = jnp.transpose(vertices, (1, 0, 2)).reshape(m, d)

    nnz = rows.shape[0]
    per_tile = -(-nnz // (_NS * _SBLEN)) * _SBLEN
    nnz_pad = per_tile * _NS
    pad = nnz_pad - nnz
    rows_p = jnp.concatenate([rows.astype(jnp.int32),
                              jnp.zeros((pad,), jnp.int32)])
    cols_p = jnp.concatenate([cols.astype(jnp.int32),
                              jnp.zeros((pad,), jnp.int32)])
    vals_p = jnp.concatenate([vals, jnp.zeros((pad,), jnp.float32)])

    out = _build(nnz_pad, n_hd, d, per_tile)(cols_p, rows_p, vals_p, m2)
    return jnp.transpose(out.reshape(n_hd, b, k), (1, 0, 2)).astype(jnp.float32)
